# baseline (speedup 1.0000x reference)
import functools

import jax
import jax.numpy as jnp
from jax import lax
from jax.experimental import pallas as pl
from jax.experimental.pallas import tpu as pltpu
from jax.experimental.pallas import tpu_sc as plsc

L = 16
K = 16
C = 512
NW = 32
QPB = 4
IDX_MASK = 0x00FFFFFF
ET_SHIFT = 24
ABS_MASK = 0x7FFFFFFF
SIGN_MASK = 0x80000000
ONE_BITS = 0x3F800000


def _sc_check_kernel(npt, input_hbm, pk_hbm, w_hbm, bb_hbm, par_hbm, out_hbm,
                     table_v, pk0, pk1, out0, out1, w_v, bb_v, par_v, tsem,
                     isem0, isem1, osem0, osem1):
    cid = lax.axis_index("c")
    sid = lax.axis_index("s")
    wid = sid * 2 + cid
    b = wid // QPB
    q = wid % QPB
    base_g = q * npt

    tcopy = pltpu.async_copy(input_hbm.at[b], table_v, tsem)
    pltpu.make_async_copy(pk_hbm.at[base_g], pk0, isem0).start()
    pltpu.sync_copy(w_hbm, w_v)
    pltpu.sync_copy(bb_hbm, bb_v)
    pltpu.sync_copy(par_hbm, par_v)
    tcopy.wait()

    alv = par_v[0, :]
    bev = par_v[1, :]
    eps = jnp.full((L,), 1e-10, jnp.float32)
    zero_u = jnp.zeros((L,), jnp.uint32)
    max_u = jnp.full((L,), 0xFFFFFFFF, jnp.uint32)
    offv = jnp.bitwise_and(lax.iota(jnp.int32, L), 3)

    def compute(buf, obuf):
        def jbody(j, c2):
            base = j * L
            sxs = [zero_u, zero_u, zero_u, zero_u]
            ams = [max_u, max_u, max_u, max_u]
            for k in range(K):
                pk = buf[k, pl.ds(base, L)]
                iv = jnp.bitwise_and(pk, IDX_MASK)
                gix = jnp.bitwise_and(lax.shift_right_logical(pk, ET_SHIFT - 2),
                                      12) + offv
                vals = plsc.load_gather(table_v, [iv])
                wv = plsc.load_gather(w_v, [gix])
                bv = plsc.load_gather(bb_v, [gix])
                v = vals * wv + bv
                u = plsc.bitcast(v + eps, jnp.uint32)
                i = k & 3
                sxs[i] = jnp.bitwise_xor(sxs[i], u)
                a = jnp.bitwise_and(plsc.bitcast(v, jnp.uint32),
                                    jnp.uint32(ABS_MASK))
                ams[i] = jnp.minimum(ams[i], a - jnp.uint32(1))
            sx = jnp.bitwise_xor(jnp.bitwise_xor(sxs[0], sxs[1]),
                                 jnp.bitwise_xor(sxs[2], sxs[3]))
            am = jnp.minimum(jnp.minimum(ams[0], ams[1]),
                             jnp.minimum(ams[2], ams[3]))
            mn = plsc.bitcast(am + jnp.uint32(1), jnp.float32)
            sgn = plsc.bitcast(
                jnp.bitwise_or(jnp.bitwise_and(sx, jnp.uint32(SIGN_MASK)),
                               jnp.uint32(ONE_BITS)),
                jnp.float32)
            obuf[pl.ds(base, L)] = sgn * (alv * mn + bev)
            return c2

        lax.fori_loop(0, C // L, jbody, 0)

    def pair_body(p, carry):
        g0 = base_g + 2 * p
        g1 = g0 + 1
        pltpu.make_async_copy(pk_hbm.at[g1], pk1, isem1).start()
        pltpu.make_async_copy(pk_hbm.at[g0], pk0, isem0).wait()

        @pl.when(p != 0)
        def _():
            pltpu.make_async_copy(out0, out_hbm.at[b, g0], osem0).wait()

        compute(pk0, out0)
        pltpu.make_async_copy(out0, out_hbm.at[b, g0], osem0).start()

        @pl.when(p != npt // 2 - 1)
        def _():
            pltpu.make_async_copy(pk_hbm.at[g0 + 2], pk0, isem0).start()

        pltpu.make_async_copy(pk_hbm.at[g1], pk1, isem1).wait()

        @pl.when(p != 0)
        def _():
            pltpu.make_async_copy(out1, out_hbm.at[b, g1], osem1).wait()

        compute(pk1, out1)
        pltpu.make_async_copy(out1, out_hbm.at[b, g1], osem1).start()
        return carry

    lax.fori_loop(0, npt // 2, pair_body, 0)
    pltpu.make_async_copy(out0, out_hbm.at[b, base_g], osem0).wait()
    pltpu.make_async_copy(out1, out_hbm.at[b, base_g], osem1).wait()


def kernel(input_tensor, check_index_tensor, edge_type_tensor, edge_weights,
           edge_biases, alpha, beta):
    B, N = input_tensor.shape
    Nm, Kk = check_index_tensor.shape
    assert Kk == K and B * QPB == NW

    step = 2 * QPB * C
    n_pad = ((Nm + step - 1) // step) * step
    nch = n_pad // C
    npt = nch // QPB

    idx32 = check_index_tensor.astype(jnp.int32)
    et32 = edge_type_tensor.astype(jnp.int32)
    packed = jnp.bitwise_or(idx32, lax.shift_left(et32, ET_SHIFT))
    pad = n_pad - Nm
    if pad:
        packed = jnp.pad(packed, ((0, pad), (0, 0)))
    pk_r = packed.reshape(nch, C, K).transpose(0, 2, 1)

    w16 = jnp.repeat(edge_weights.astype(jnp.float32), 4)[:L]
    b16 = jnp.repeat(edge_biases.astype(jnp.float32), 4)[:L]
    params = jnp.concatenate([alpha.astype(jnp.float32),
                              beta.astype(jnp.float32)])
    par = jnp.broadcast_to(params[:, None], (2, L))

    mesh = plsc.VectorSubcoreMesh(core_axis_name="c", subcore_axis_name="s")
    fn = pl.kernel(
        functools.partial(_sc_check_kernel, npt),
        mesh=mesh,
        compiler_params=pltpu.CompilerParams(needs_layout_passes=False),
        out_type=jax.ShapeDtypeStruct((B, nch, C), jnp.float32),
        scratch_types=[
            pltpu.VMEM((N,), jnp.float32),
            pltpu.VMEM((K, C), jnp.int32),
            pltpu.VMEM((K, C), jnp.int32),
            pltpu.VMEM((C,), jnp.float32),
            pltpu.VMEM((C,), jnp.float32),
            pltpu.VMEM((L,), jnp.float32),
            pltpu.VMEM((L,), jnp.float32),
            pltpu.VMEM((2, L), jnp.float32),
            pltpu.SemaphoreType.DMA,
            pltpu.SemaphoreType.DMA,
            pltpu.SemaphoreType.DMA,
            pltpu.SemaphoreType.DMA,
            pltpu.SemaphoreType.DMA,
        ],
    )
    out = fn(input_tensor, pk_r, w16, b16, par)
    return out.reshape(B, n_pad)[:, :Nm]


# speedup vs baseline: 1.0789x; 1.0789x over previous
import functools

import jax
import jax.numpy as jnp
from jax import lax
from jax.experimental import pallas as pl
from jax.experimental.pallas import tpu as pltpu
from jax.experimental.pallas import tpu_sc as plsc

L = 16
K = 16
C = 512
NW = 32
QPB = 4
IDX_MASK = 0x00FFFFFF
ET_SHIFT = 24
ABS_MASK = 0x7FFFFFFF
SIGN_MASK = 0x80000000
ONE_BITS = 0x3F800000


def _sc_check_kernel(npt, input_hbm, pk_hbm, w_hbm, bb_hbm, par_hbm, out_hbm,
                     table_v, pk0, pk1, out0, out1, w_v, bb_v, par_v, tsem,
                     isem0, isem1, osem0, osem1):
    cid = lax.axis_index("c")
    sid = lax.axis_index("s")
    wid = sid * 2 + cid
    b = wid // QPB
    q = wid % QPB
    base_g = q * npt

    tcopy = pltpu.async_copy(input_hbm.at[b], table_v, tsem)
    pltpu.make_async_copy(pk_hbm.at[base_g], pk0, isem0).start()
    pltpu.sync_copy(w_hbm, w_v)
    pltpu.sync_copy(bb_hbm, bb_v)
    pltpu.sync_copy(par_hbm, par_v)
    tcopy.wait()

    alv = par_v[0, :]
    bev = par_v[1, :]
    eps = jnp.full((L,), 1e-10, jnp.float32)
    zero_u = jnp.zeros((L,), jnp.uint32)
    max_u = jnp.full((L,), 0xFFFFFFFF, jnp.uint32)
    offv = jnp.bitwise_and(lax.iota(jnp.int32, L), 3)

    def compute(buf, obuf):
        def jbody(j, c2):
          for h in range(2):
            base = (2 * j + h) * L
            sxs = [zero_u, zero_u, zero_u, zero_u]
            ams = [max_u, max_u, max_u, max_u]
            for k in range(K):
                pk = buf[k, pl.ds(base, L)]
                iv = jnp.bitwise_and(pk, IDX_MASK)
                gix = jnp.bitwise_and(lax.shift_right_logical(pk, ET_SHIFT - 2),
                                      12) + offv
                vals = plsc.load_gather(table_v, [iv])
                wv = plsc.load_gather(w_v, [gix])
                bv = plsc.load_gather(bb_v, [gix])
                v = vals * wv + bv
                u = plsc.bitcast(v + eps, jnp.uint32)
                i = k & 3
                sxs[i] = jnp.bitwise_xor(sxs[i], u)
                a = jnp.bitwise_and(plsc.bitcast(v, jnp.uint32),
                                    jnp.uint32(ABS_MASK))
                ams[i] = jnp.minimum(ams[i], a - jnp.uint32(1))
            sx = jnp.bitwise_xor(jnp.bitwise_xor(sxs[0], sxs[1]),
                                 jnp.bitwise_xor(sxs[2], sxs[3]))
            am = jnp.minimum(jnp.minimum(ams[0], ams[1]),
                             jnp.minimum(ams[2], ams[3]))
            mn = plsc.bitcast(am + jnp.uint32(1), jnp.float32)
            sgn = plsc.bitcast(
                jnp.bitwise_or(jnp.bitwise_and(sx, jnp.uint32(SIGN_MASK)),
                               jnp.uint32(ONE_BITS)),
                jnp.float32)
            obuf[pl.ds(base, L)] = sgn * (alv * mn + bev)
          return c2

        lax.fori_loop(0, C // (2 * L), jbody, 0)

    def pair_body(p, carry):
        g0 = base_g + 2 * p
        g1 = g0 + 1
        pltpu.make_async_copy(pk_hbm.at[g1], pk1, isem1).start()
        pltpu.make_async_copy(pk_hbm.at[g0], pk0, isem0).wait()

        @pl.when(p != 0)
        def _():
            pltpu.make_async_copy(out0, out_hbm.at[b, g0], osem0).wait()

        compute(pk0, out0)
        pltpu.make_async_copy(out0, out_hbm.at[b, g0], osem0).start()

        @pl.when(p != npt // 2 - 1)
        def _():
            pltpu.make_async_copy(pk_hbm.at[g0 + 2], pk0, isem0).start()

        pltpu.make_async_copy(pk_hbm.at[g1], pk1, isem1).wait()

        @pl.when(p != 0)
        def _():
            pltpu.make_async_copy(out1, out_hbm.at[b, g1], osem1).wait()

        compute(pk1, out1)
        pltpu.make_async_copy(out1, out_hbm.at[b, g1], osem1).start()
        return carry

    lax.fori_loop(0, npt // 2, pair_body, 0)
    pltpu.make_async_copy(out0, out_hbm.at[b, base_g], osem0).wait()
    pltpu.make_async_copy(out1, out_hbm.at[b, base_g], osem1).wait()


def kernel(input_tensor, check_index_tensor, edge_type_tensor, edge_weights,
           edge_biases, alpha, beta):
    B, N = input_tensor.shape
    Nm, Kk = check_index_tensor.shape
    assert Kk == K and B * QPB == NW

    step = 2 * QPB * C
    n_pad = ((Nm + step - 1) // step) * step
    nch = n_pad // C
    npt = nch // QPB

    idx32 = check_index_tensor.astype(jnp.int32)
    et32 = edge_type_tensor.astype(jnp.int32)
    packed = jnp.bitwise_or(idx32, lax.shift_left(et32, ET_SHIFT))
    pad = n_pad - Nm
    if pad:
        packed = jnp.pad(packed, ((0, pad), (0, 0)))
    pk_r = packed.reshape(nch, C, K).transpose(0, 2, 1)

    w16 = jnp.repeat(edge_weights.astype(jnp.float32), 4)[:L]
    b16 = jnp.repeat(edge_biases.astype(jnp.float32), 4)[:L]
    params = jnp.concatenate([alpha.astype(jnp.float32),
                              beta.astype(jnp.float32)])
    par = jnp.broadcast_to(params[:, None], (2, L))

    mesh = plsc.VectorSubcoreMesh(core_axis_name="c", subcore_axis_name="s")
    fn = pl.kernel(
        functools.partial(_sc_check_kernel, npt),
        mesh=mesh,
        compiler_params=pltpu.CompilerParams(needs_layout_passes=False),
        out_type=jax.ShapeDtypeStruct((B, nch, C), jnp.float32),
        scratch_types=[
            pltpu.VMEM((N,), jnp.float32),
            pltpu.VMEM((K, C), jnp.int32),
            pltpu.VMEM((K, C), jnp.int32),
            pltpu.VMEM((C,), jnp.float32),
            pltpu.VMEM((C,), jnp.float32),
            pltpu.VMEM((L,), jnp.float32),
            pltpu.VMEM((L,), jnp.float32),
            pltpu.VMEM((2, L), jnp.float32),
            pltpu.SemaphoreType.DMA,
            pltpu.SemaphoreType.DMA,
            pltpu.SemaphoreType.DMA,
            pltpu.SemaphoreType.DMA,
            pltpu.SemaphoreType.DMA,
        ],
    )
    out = fn(input_tensor, pk_r, w16, b16, par)
    return out.reshape(B, n_pad)[:, :Nm]


# speedup vs baseline: 1.0853x; 1.0059x over previous
import functools

import jax
import jax.numpy as jnp
from jax import lax
from jax.experimental import pallas as pl
from jax.experimental.pallas import tpu as pltpu
from jax.experimental.pallas import tpu_sc as plsc

L = 16
K = 16
C = 512
NW = 32
QPB = 4
IDX_MASK = 0x00FFFFFF
ET_SHIFT = 24
ABS_MASK = 0x7FFFFFFF
SIGN_MASK = 0x80000000
ONE_BITS = 0x3F800000


def _sc_check_kernel(npt, input_hbm, pk_hbm, w_hbm, bb_hbm, par_hbm, out_hbm,
                     table_v, pk0, pk1, out0, out1, w_v, bb_v, par_v, tsem,
                     isem0, isem1, osem0, osem1):
    cid = lax.axis_index("c")
    sid = lax.axis_index("s")
    wid = sid * 2 + cid
    b = wid // QPB
    q = wid % QPB
    base_g = q * npt

    tcopy = pltpu.async_copy(input_hbm.at[b], table_v, tsem)
    pltpu.make_async_copy(pk_hbm.at[base_g], pk0, isem0).start()
    pltpu.sync_copy(w_hbm, w_v)
    pltpu.sync_copy(bb_hbm, bb_v)
    pltpu.sync_copy(par_hbm, par_v)
    tcopy.wait()

    alv = par_v[0, :]
    bev = par_v[1, :]
    eps = jnp.full((L,), 1e-10, jnp.float32)
    zero_u = jnp.zeros((L,), jnp.uint32)
    max_u = jnp.full((L,), 0xFFFFFFFF, jnp.uint32)
    offv = jnp.bitwise_and(lax.iota(jnp.int32, L), 3)

    def compute(buf, obuf):
        def jbody(j, c2):
          for h in range(4):
            base = (4 * j + h) * L
            sxs = [zero_u, zero_u, zero_u, zero_u]
            ams = [max_u, max_u, max_u, max_u]
            for k in range(K):
                pk = buf[k, pl.ds(base, L)]
                iv = jnp.bitwise_and(pk, IDX_MASK)
                gix = jnp.bitwise_and(lax.shift_right_logical(pk, ET_SHIFT - 2),
                                      12) + offv
                vals = plsc.load_gather(table_v, [iv])
                wv = plsc.load_gather(w_v, [gix])
                bv = plsc.load_gather(bb_v, [gix])
                v = vals * wv + bv
                u = plsc.bitcast(v + eps, jnp.uint32)
                i = k & 3
                sxs[i] = jnp.bitwise_xor(sxs[i], u)
                a = jnp.bitwise_and(plsc.bitcast(v, jnp.uint32),
                                    jnp.uint32(ABS_MASK))
                ams[i] = jnp.minimum(ams[i], a - jnp.uint32(1))
            sx = jnp.bitwise_xor(jnp.bitwise_xor(sxs[0], sxs[1]),
                                 jnp.bitwise_xor(sxs[2], sxs[3]))
            am = jnp.minimum(jnp.minimum(ams[0], ams[1]),
                             jnp.minimum(ams[2], ams[3]))
            mn = plsc.bitcast(am + jnp.uint32(1), jnp.float32)
            sgn = plsc.bitcast(
                jnp.bitwise_or(jnp.bitwise_and(sx, jnp.uint32(SIGN_MASK)),
                               jnp.uint32(ONE_BITS)),
                jnp.float32)
            obuf[pl.ds(base, L)] = sgn * (alv * mn + bev)
          return c2

        lax.fori_loop(0, C // (4 * L), jbody, 0)

    def pair_body(p, carry):
        g0 = base_g + 2 * p
        g1 = g0 + 1
        pltpu.make_async_copy(pk_hbm.at[g1], pk1, isem1).start()
        pltpu.make_async_copy(pk_hbm.at[g0], pk0, isem0).wait()

        @pl.when(p != 0)
        def _():
            pltpu.make_async_copy(out0, out_hbm.at[b, g0], osem0).wait()

        compute(pk0, out0)
        pltpu.make_async_copy(out0, out_hbm.at[b, g0], osem0).start()

        @pl.when(p != npt // 2 - 1)
        def _():
            pltpu.make_async_copy(pk_hbm.at[g0 + 2], pk0, isem0).start()

        pltpu.make_async_copy(pk_hbm.at[g1], pk1, isem1).wait()

        @pl.when(p != 0)
        def _():
            pltpu.make_async_copy(out1, out_hbm.at[b, g1], osem1).wait()

        compute(pk1, out1)
        pltpu.make_async_copy(out1, out_hbm.at[b, g1], osem1).start()
        return carry

    lax.fori_loop(0, npt // 2, pair_body, 0)
    pltpu.make_async_copy(out0, out_hbm.at[b, base_g], osem0).wait()
    pltpu.make_async_copy(out1, out_hbm.at[b, base_g], osem1).wait()


def kernel(input_tensor, check_index_tensor, edge_type_tensor, edge_weights,
           edge_biases, alpha, beta):
    B, N = input_tensor.shape
    Nm, Kk = check_index_tensor.shape
    assert Kk == K and B * QPB == NW

    step = 2 * QPB * C
    n_pad = ((Nm + step - 1) // step) * step
    nch = n_pad // C
    npt = nch // QPB

    idx32 = check_index_tensor.astype(jnp.int32)
    et32 = edge_type_tensor.astype(jnp.int32)
    packed = jnp.bitwise_or(idx32, lax.shift_left(et32, ET_SHIFT))
    pad = n_pad - Nm
    if pad:
        packed = jnp.pad(packed, ((0, pad), (0, 0)))
    pk_r = packed.reshape(nch, C, K).transpose(0, 2, 1)

    w16 = jnp.repeat(edge_weights.astype(jnp.float32), 4)[:L]
    b16 = jnp.repeat(edge_biases.astype(jnp.float32), 4)[:L]
    params = jnp.concatenate([alpha.astype(jnp.float32),
                              beta.astype(jnp.float32)])
    par = jnp.broadcast_to(params[:, None], (2, L))

    mesh = plsc.VectorSubcoreMesh(core_axis_name="c", subcore_axis_name="s")
    fn = pl.kernel(
        functools.partial(_sc_check_kernel, npt),
        mesh=mesh,
        compiler_params=pltpu.CompilerParams(needs_layout_passes=False),
        out_type=jax.ShapeDtypeStruct((B, nch, C), jnp.float32),
        scratch_types=[
            pltpu.VMEM((N,), jnp.float32),
            pltpu.VMEM((K, C), jnp.int32),
            pltpu.VMEM((K, C), jnp.int32),
            pltpu.VMEM((C,), jnp.float32),
            pltpu.VMEM((C,), jnp.float32),
            pltpu.VMEM((L,), jnp.float32),
            pltpu.VMEM((L,), jnp.float32),
            pltpu.VMEM((2, L), jnp.float32),
            pltpu.SemaphoreType.DMA,
            pltpu.SemaphoreType.DMA,
            pltpu.SemaphoreType.DMA,
            pltpu.SemaphoreType.DMA,
            pltpu.SemaphoreType.DMA,
        ],
    )
    out = fn(input_tensor, pk_r, w16, b16, par)
    return out.reshape(B, n_pad)[:, :Nm]
